# Initial kernel scaffold; baseline (speedup 1.0000x reference)
import jax
import jax.numpy as jnp
from jax.experimental import pallas as pl


def kernel(x, W_enc, b_enc, W_dec):
    raise NotImplementedError("write your pallas kernel here")


# speedup vs baseline: 1.3724x; 1.3724x over previous
import jax
import jax.numpy as jnp
from jax.experimental import pallas as pl
from jax.experimental.pallas import tpu as pltpu

_INPUT = 2048
_HIDDEN = 16384
_K = 32
_B = 32
_BLK = 2048
_NBLK = _HIDDEN // _BLK


def _enc_body(x_ref, w_ref, b_ref, h_ref):
    acc = jax.lax.dot_general(
        x_ref[...], w_ref[...],
        (((1,), (1,)), ((), ())),
        preferred_element_type=jnp.float32,
    )
    h_ref[...] = jnp.maximum(acc + b_ref[...], 0.0)


def _dec_body(h_ref, w_ref, o_ref, t_ref, it_ref, hw_ref):
    i = pl.program_id(0)

    @pl.when(i == 0)
    def _topk():
        hw_ref[...] = h_ref[...]

        def body(j, carry):
            hw = hw_ref[...]
            m = jnp.max(hw, axis=1, keepdims=True)
            iota = jax.lax.broadcasted_iota(jnp.int32, hw.shape, 1)
            im = jnp.min(jnp.where(hw == m, iota, _HIDDEN),
                         axis=1, keepdims=True)
            hw_ref[...] = jnp.where(iota == im, -1.0, hw)
            t_ref[...] = m
            it_ref[...] = im
            return carry

        jax.lax.fori_loop(0, _K, body, 0)

    hblk = h_ref[:, pl.ds(i * _BLK, _BLK)]
    iota = jax.lax.broadcasted_iota(jnp.int32, (_B, _BLK), 1) + i * _BLK
    t = t_ref[...]
    it = it_ref[...]
    keep = (hblk > t) | ((hblk == t) & (iota <= it))
    hs = jnp.where(keep, hblk, 0.0)
    acc = jax.lax.dot_general(
        hs, w_ref[...],
        (((1,), (1,)), ((), ())),
        preferred_element_type=jnp.float32,
    )

    @pl.when(i == 0)
    def _init():
        o_ref[...] = acc

    @pl.when(i > 0)
    def _acc():
        o_ref[...] += acc


def kernel(x, W_enc, b_enc, W_dec):
    b2 = b_enc.reshape(1, _HIDDEN)

    h = pl.pallas_call(
        _enc_body,
        grid=(_NBLK,),
        in_specs=[
            pl.BlockSpec((_B, _INPUT), lambda i: (0, 0)),
            pl.BlockSpec((_BLK, _INPUT), lambda i: (i, 0)),
            pl.BlockSpec((1, _BLK), lambda i: (0, i)),
        ],
        out_specs=pl.BlockSpec((_B, _BLK), lambda i: (0, i)),
        out_shape=jax.ShapeDtypeStruct((_B, _HIDDEN), jnp.float32),
    )(x, W_enc, b2)

    recon = pl.pallas_call(
        _dec_body,
        grid=(_NBLK,),
        in_specs=[
            pl.BlockSpec((_B, _HIDDEN), lambda i: (0, 0)),
            pl.BlockSpec((_INPUT, _BLK), lambda i: (0, i)),
        ],
        out_specs=pl.BlockSpec((_B, _INPUT), lambda i: (0, 0)),
        out_shape=jax.ShapeDtypeStruct((_B, _INPUT), jnp.float32),
        scratch_shapes=[
            pltpu.VMEM((_B, 1), jnp.float32),
            pltpu.VMEM((_B, 1), jnp.int32),
            pltpu.VMEM((_B, _HIDDEN), jnp.float32),
        ],
    )(h, W_dec)

    return recon


# speedup vs baseline: 1.4027x; 1.0220x over previous
import jax
import jax.numpy as jnp
from jax.experimental import pallas as pl
from jax.experimental.pallas import tpu as pltpu

_INPUT = 2048
_HIDDEN = 16384
_K = 32
_B = 32
_BLK = 2048
_NBLK = _HIDDEN // _BLK


def _enc_body(x_ref, w_ref, b_ref, h_ref):
    acc = jax.lax.dot_general(
        x_ref[...], w_ref[...],
        (((1,), (1,)), ((), ())),
        preferred_element_type=jnp.float32,
    )
    h_ref[...] = jnp.maximum(acc + b_ref[...], 0.0)


_NCHUNK = 128
_CW = _HIDDEN // _NCHUNK
_NSLOT = 8


def _dec_body(h_ref, w_ref, o_ref, t_ref, it_ref, hw_ref):
    i = pl.program_id(0)

    @pl.when(i == 0)
    def _topk():
        hw3 = h_ref[...].reshape(_B, _NCHUNK, _CW)
        iota_e = jax.lax.broadcasted_iota(jnp.int32, (_B, _NCHUNK, _CW), 2)
        iota_c = jax.lax.broadcasted_iota(jnp.int32, (_B, _NCHUNK), 1)
        cvs = []
        cgs = []
        for _ in range(_NSLOT):
            cm = jnp.max(hw3, axis=2)
            im = jnp.min(jnp.where(hw3 == cm[:, :, None], iota_e, _CW),
                         axis=2)
            hw3 = jnp.where(iota_e == im[:, :, None], -1.0, hw3)
            cvs.append(cm)
            cgs.append(iota_c * _CW + im)
        Cv0 = jnp.stack(cvs, axis=1)
        Cg0 = jnp.stack(cgs, axis=1)

        def cbody(j, carry):
            Cv, Cg = carry
            m = jnp.max(Cv, axis=(1, 2), keepdims=True)
            gi = jnp.min(jnp.where(Cv == m, Cg, _HIDDEN),
                         axis=(1, 2), keepdims=True)
            Cv = jnp.where((Cv == m) & (Cg == gi), -1.0, Cv)
            t_ref[...] = m.reshape(_B, 1)
            it_ref[...] = gi.reshape(_B, 1)
            return (Cv, Cg)

        jax.lax.fori_loop(0, _K, cbody, (Cv0, Cg0))

        h = h_ref[...]
        iota = jax.lax.broadcasted_iota(jnp.int32, (_B, _HIDDEN), 1)
        keep = (h > t_ref[...]) | ((h == t_ref[...]) & (iota <= it_ref[...]))
        cnt = jnp.sum(keep.astype(jnp.int32), axis=1)
        bad = jnp.any(cnt != _K)

        @pl.when(bad)
        def _fallback():
            hw_ref[...] = h_ref[...]

            def body(j, carry):
                hw = hw_ref[...]
                m = jnp.max(hw, axis=1, keepdims=True)
                im = jnp.min(jnp.where(hw == m, iota, _HIDDEN),
                             axis=1, keepdims=True)
                hw_ref[...] = jnp.where(iota == im, -1.0, hw)
                t_ref[...] = m
                it_ref[...] = im
                return carry

            jax.lax.fori_loop(0, _K, body, 0)

    hblk = h_ref[:, pl.ds(i * _BLK, _BLK)]
    iota = jax.lax.broadcasted_iota(jnp.int32, (_B, _BLK), 1) + i * _BLK
    t = t_ref[...]
    it = it_ref[...]
    keep = (hblk > t) | ((hblk == t) & (iota <= it))
    hs = jnp.where(keep, hblk, 0.0)
    acc = jax.lax.dot_general(
        hs, w_ref[...],
        (((1,), (1,)), ((), ())),
        preferred_element_type=jnp.float32,
    )

    @pl.when(i == 0)
    def _init():
        o_ref[...] = acc

    @pl.when(i > 0)
    def _acc():
        o_ref[...] += acc


def kernel(x, W_enc, b_enc, W_dec):
    b2 = b_enc.reshape(1, _HIDDEN)

    h = pl.pallas_call(
        _enc_body,
        grid=(_NBLK,),
        in_specs=[
            pl.BlockSpec((_B, _INPUT), lambda i: (0, 0)),
            pl.BlockSpec((_BLK, _INPUT), lambda i: (i, 0)),
            pl.BlockSpec((1, _BLK), lambda i: (0, i)),
        ],
        out_specs=pl.BlockSpec((_B, _BLK), lambda i: (0, i)),
        out_shape=jax.ShapeDtypeStruct((_B, _HIDDEN), jnp.float32),
    )(x, W_enc, b2)

    recon = pl.pallas_call(
        _dec_body,
        grid=(_NBLK,),
        in_specs=[
            pl.BlockSpec((_B, _HIDDEN), lambda i: (0, 0)),
            pl.BlockSpec((_INPUT, _BLK), lambda i: (0, i)),
        ],
        out_specs=pl.BlockSpec((_B, _INPUT), lambda i: (0, 0)),
        out_shape=jax.ShapeDtypeStruct((_B, _INPUT), jnp.float32),
        scratch_shapes=[
            pltpu.VMEM((_B, 1), jnp.float32),
            pltpu.VMEM((_B, 1), jnp.int32),
            pltpu.VMEM((_B, _HIDDEN), jnp.float32),
        ],
    )(h, W_dec)

    return recon
